# baseline (speedup 1.0000x reference)
import functools

import numpy as np
import jax
import jax.numpy as jnp
from jax import lax
from jax.experimental import pallas as pl
from jax.experimental.pallas import tpu as pltpu
import jax.experimental.pallas.tpu_sc as plsc

_PI = np.pi
_X_MIN, _X_N = 1e-05, 5000
_SIGMA_MIN, _SIGMA_MAX, _SIGMA_N = 0.003, 2, 5000

_N = 1_000_000
_M = 4 * _N
_ROWS = _M // 128
_RB = 250
_G = _ROWS // _RB

_NC, _NS = 2, 16
_NW = _NC * _NS
_CH = 16
_NROUNDS = _ROWS // (_CH * _NW)
_R_EXTRA = _NROUNDS * _CH * _NW
_R_TAIL = _R_EXTRA + _CH
_TAIL = _ROWS - _R_TAIL


_F32 = np.float32
_C_PI = _F32(np.pi)
_C_2PI = _F32(2 * np.pi)
_C_INVPI = _F32(1.0) / _C_PI
_C_EPS = _F32(1e-10)
_C_XOFF = _F32(-np.log(_X_MIN))
_C_XSCL = (_F32(1.0) / _F32(0 - np.log(_X_MIN))) * _F32(_X_N)
_C_SOFF = _F32(-np.log(_SIGMA_MIN))
_C_SSCL = (_F32(1.0) / _F32(np.log(_SIGMA_MAX) - np.log(_SIGMA_MIN))) * _F32(_SIGMA_N)


def _index_body(x_ref, s_ref, lin_ref, sgn_ref):
    x = x_ref[...]
    sg = s_ref[...]
    r = lax.rem(x + _C_PI, jnp.full_like(x, _C_2PI))
    r = jnp.where((r < 0) & (r != 0), r + _C_2PI, r)
    xw = r + (-_C_PI)
    sign = jnp.sign(xw)
    xl = jnp.log(jnp.abs(xw) * _C_INVPI + _C_EPS)
    xi = (xl + _C_XOFF) * _C_XSCL
    xi = jnp.round(jnp.clip(xi, 0, _X_N)).astype(jnp.int32)
    sl = jnp.log(sg * _C_INVPI)
    si = (sl + _C_SOFF) * _C_SSCL
    si = jnp.round(jnp.clip(si, 0, _SIGMA_N)).astype(jnp.int32)
    lin_ref[...] = si * (_X_N + 1) + xi
    sgn_ref[...] = -sign


def _index_pallas(x3, s3):
    return pl.pallas_call(
        _index_body,
        grid=(_G,),
        in_specs=[
            pl.BlockSpec((1, _RB, 128), lambda i: (i, 0, 0)),
            pl.BlockSpec((1, _RB, 128), lambda i: (i, 0, 0)),
        ],
        out_specs=[
            pl.BlockSpec((1, _RB, 128), lambda i: (i, 0, 0)),
            pl.BlockSpec((1, _RB, 128), lambda i: (i, 0, 0)),
        ],
        out_shape=[
            jax.ShapeDtypeStruct((_G, _RB, 128), jnp.int32),
            jax.ShapeDtypeStruct((_G, _RB, 128), jnp.float32),
        ],
    )(x3, s3)


def _gather_body(tab_ref, lin_ref, sgn_ref, out_ref, idx_v, sgn_v, val_v, gsem):
    w = lax.axis_index("s") * _NC + lax.axis_index("c")

    def mul_row(j, k):
        val_v[j, pl.ds(k * 16, 16)] = (
            val_v[j, pl.ds(k * 16, 16)] * sgn_v[j, pl.ds(k * 16, 16)]
        )

    def process(r0, nr):
        pltpu.sync_copy(lin_ref.at[pl.ds(r0, nr)], idx_v.at[pl.ds(0, nr)])
        pltpu.sync_copy(sgn_ref.at[pl.ds(r0, nr)], sgn_v.at[pl.ds(0, nr)])
        cps = [
            pltpu.async_copy(tab_ref.at[idx_v.at[j]], val_v.at[j], gsem)
            for j in range(nr)
        ]
        for cp in cps:
            cp.wait()

        def mul_body(k, carry2):
            for j in range(nr):
                mul_row(j, k)
            return carry2

        lax.fori_loop(0, 8, mul_body, 0)
        pltpu.sync_copy(val_v.at[pl.ds(0, nr)], out_ref.at[pl.ds(r0, nr)])

    def chunk_body(c, carry):
        process((c * _NW + w) * _CH, _CH)
        return carry

    lax.fori_loop(0, _NROUNDS, chunk_body, 0)

    @pl.when(w == 0)
    def _():
        process(_R_EXTRA, _CH)

    @pl.when(w == 1)
    def _():
        process(_R_TAIL, _TAIL)


def _gather_pallas(tab, lin2, sgn2):
    mesh = plsc.VectorSubcoreMesh(
        core_axis_name="c", subcore_axis_name="s",
        num_cores=_NC, num_subcores=_NS,
    )
    return pl.kernel(
        _gather_body,
        out_type=jax.ShapeDtypeStruct((_ROWS, 128), jnp.float32),
        mesh=mesh,
        scratch_types=[
            pltpu.VMEM((_CH, 128), jnp.int32),
            pltpu.VMEM((_CH, 128), jnp.float32),
            pltpu.VMEM((_CH, 128), jnp.float32),
            pltpu.SemaphoreType.DMA,
        ],
    )(tab, lin2, sgn2)


def kernel(x, sigma, score_table):
    x3 = x.reshape(_G, _RB, 128)
    s3 = jnp.broadcast_to(sigma, (_N, 4)).reshape(_G, _RB, 128)
    lin3, sgn3 = _index_pallas(x3, s3)
    tab = score_table.reshape(-1)
    out2 = _gather_pallas(tab, lin3.reshape(_ROWS, 128), sgn3.reshape(_ROWS, 128))
    return out2.reshape(_N, 4)
